# baseline (speedup 1.0000x reference)
import jax
import jax.numpy as jnp
from jax import lax
from jax.experimental import pallas as pl
from jax.experimental.pallas import tpu as pltpu
from jax.experimental.pallas import tpu_sc as plsc

B = 64
L = 2048
D = 64
DP = 128
V = 1000000
NC = 2
NS = 16
NW = NC * NS
CL = 128
NBG = 8
UNITS = (B // NBG) * (L // CL)
UPW = UNITS // NW


def _body(x_hbm, tab_hbm, posT_hbm, out_hbm, idx_v, posT_v, gbuf_v, ob_v, sem):
    wid = lax.axis_index("s") * NC + lax.axis_index("c")
    iota = jax.lax.iota(jnp.int32, 16)
    jconsts = [iota + 16 * k for k in range(CL // 16)]

    for u in range(UPW):
        unit = wid * UPW + u
        bg = unit >> 4
        lc = unit & 15
        l0 = pl.multiple_of(lc * CL, CL)
        pltpu.sync_copy(posT_hbm.at[:, pl.ds(l0, CL)], posT_v)
        for b in range(NBG):
            row = bg * NBG + b
            pltpu.sync_copy(x_hbm.at[pl.ds(row * L + l0, CL)], idx_v)
            pltpu.async_copy(tab_hbm.at[idx_v], gbuf_v, sem).wait()

            def d_step(d, carry):
                for k in range(CL // 16):
                    val = plsc.load_gather(
                        gbuf_v, [jconsts[k], jnp.zeros((16,), jnp.int32) + d]
                    )
                    ob_v[d, pl.ds(16 * k, 16)] = val + posT_v[d, pl.ds(16 * k, 16)]
                return carry

            lax.fori_loop(0, D, d_step, 0)
            pltpu.sync_copy(ob_v, out_hbm.at[row, :, pl.ds(l0, CL)])


@jax.jit
def kernel(x, token_embedding, pos_encoding):
    x1d = x.astype(jnp.int32).reshape(B * L)
    tab_p = jnp.pad(token_embedding, ((0, 0), (0, DP - D)))
    posT = pos_encoding.T
    mesh = plsc.VectorSubcoreMesh(core_axis_name="c", subcore_axis_name="s")
    out = pl.kernel(
        _body,
        out_type=jax.ShapeDtypeStruct((B, D, L), jnp.float32),
        mesh=mesh,
        scratch_types=[
            pltpu.VMEM((CL,), jnp.int32),
            pltpu.VMEM((D, CL), jnp.float32),
            pltpu.VMEM((CL, DP), jnp.float32),
            pltpu.VMEM((D, CL), jnp.float32),
            pltpu.SemaphoreType.DMA,
        ],
        compiler_params=pltpu.CompilerParams(
            use_tc_tiling_on_sc=False, needs_layout_passes=False
        ),
    )(x1d, tab_p, posT)
    return out.transpose(0, 2, 1)


# speedup vs baseline: 1.1689x; 1.1689x over previous
import functools

import jax
import jax.numpy as jnp
from jax import lax
from jax.experimental import pallas as pl
from jax.experimental.pallas import tpu as pltpu
from jax.experimental.pallas import tpu_sc as plsc

B = 64
L = 2048
D = 64
NC = 2
NS = 16
NW = NC * NS
BPW = B // NW
C = 512
G = 128
NG = C // G
NCHUNK = L // C


def _body(x_hbm, tab_hbm, pos_hbm, out_hbm, idx_v, pos_v, rows_v, sem):
    wid = lax.axis_index("s") * NC + lax.axis_index("c")

    def chunk_step(c, carry):
        pltpu.sync_copy(pos_hbm.at[pl.ds(c * C, C), :], pos_v)
        for b in range(BPW):
            row0 = (wid * BPW + b) * L + c * C
            pltpu.sync_copy(x_hbm.at[pl.ds(row0, C)], idx_v)
            copies = [
                pltpu.async_copy(
                    tab_hbm.at[idx_v.at[pl.ds(j * G, G)]],
                    rows_v.at[pl.ds(j * G, G), :],
                    sem,
                )
                for j in range(NG)
            ]
            for cp in copies:
                cp.wait()

            def add_step(i, carry2):
                for u in range(2):
                    r = i * 2 + u
                    for d in range(D // 16):
                        plsc.addupdate(
                            rows_v.at[r, pl.ds(d * 16, 16)],
                            pos_v[r, pl.ds(d * 16, 16)],
                        )
                return carry2

            lax.fori_loop(0, C // 2, add_step, 0)
            pltpu.sync_copy(rows_v, out_hbm.at[pl.ds(row0, C), :])
        return carry

    lax.fori_loop(0, NCHUNK, chunk_step, 0)


@jax.jit
def kernel(x, token_embedding, pos_encoding):
    x1d = x.astype(jnp.int32).reshape(B * L)
    mesh = plsc.VectorSubcoreMesh(core_axis_name="c", subcore_axis_name="s")
    out = pl.kernel(
        _body,
        out_type=jax.ShapeDtypeStruct((B * L, D), jnp.float32),
        mesh=mesh,
        scratch_types=[
            pltpu.VMEM((C,), jnp.int32),
            pltpu.VMEM((C, D), jnp.float32),
            pltpu.VMEM((C, D), jnp.float32),
            pltpu.SemaphoreType.DMA,
        ],
        compiler_params=pltpu.CompilerParams(use_tc_tiling_on_sc=False),
    )(x1d, token_embedding, pos_encoding)
    return out.reshape(B, L, D)


# speedup vs baseline: 1.1709x; 1.0017x over previous
import functools

import jax
import jax.numpy as jnp
from jax import lax
from jax.experimental import pallas as pl
from jax.experimental.pallas import tpu as pltpu
from jax.experimental.pallas import tpu_sc as plsc

B = 64
L = 2048
D = 64
NC = 2
NS = 16
NW = NC * NS
BPW = B // NW
C = 512
G = 512
NG = C // G
NCHUNK = L // C


def _body(x_hbm, tab_hbm, pos_hbm, out_hbm, idx_v, pos_v, rows_v, sem):
    wid = lax.axis_index("s") * NC + lax.axis_index("c")

    def chunk_step(c, carry):
        pltpu.sync_copy(pos_hbm.at[pl.ds(c * C, C), :], pos_v)
        for b in range(BPW):
            row0 = (wid * BPW + b) * L + c * C
            pltpu.sync_copy(x_hbm.at[pl.ds(row0, C)], idx_v)
            copies = [
                pltpu.async_copy(
                    tab_hbm.at[idx_v.at[pl.ds(j * G, G)]],
                    rows_v.at[pl.ds(j * G, G), :],
                    sem,
                )
                for j in range(NG)
            ]
            for cp in copies:
                cp.wait()

            def add_step(i, carry2):
                for u in range(2):
                    r = i * 2 + u
                    for d in range(D // 16):
                        plsc.addupdate(
                            rows_v.at[r, pl.ds(d * 16, 16)],
                            pos_v[r, pl.ds(d * 16, 16)],
                        )
                return carry2

            lax.fori_loop(0, C // 2, add_step, 0)
            pltpu.sync_copy(rows_v, out_hbm.at[pl.ds(row0, C), :])
        return carry

    lax.fori_loop(0, NCHUNK, chunk_step, 0)


@jax.jit
def kernel(x, token_embedding, pos_encoding):
    x1d = x.astype(jnp.int32).reshape(B * L)
    mesh = plsc.VectorSubcoreMesh(core_axis_name="c", subcore_axis_name="s")
    out = pl.kernel(
        _body,
        out_type=jax.ShapeDtypeStruct((B * L, D), jnp.float32),
        mesh=mesh,
        scratch_types=[
            pltpu.VMEM((C,), jnp.int32),
            pltpu.VMEM((C, D), jnp.float32),
            pltpu.VMEM((C, D), jnp.float32),
            pltpu.SemaphoreType.DMA,
        ],
        compiler_params=pltpu.CompilerParams(use_tc_tiling_on_sc=False),
    )(x1d, token_embedding, pos_encoding)
    return out.reshape(B, L, D)
